# baseline (speedup 1.0000x reference)
import functools
import jax
import jax.numpy as jnp
from jax import lax
from jax.experimental import pallas as pl
from jax.experimental.pallas import tpu as pltpu
from jax.experimental.pallas import tpu_sc as plsc

H = 768
L = 16
NC, NS = 2, 16
NW = NC * NS
HC = H // L
EPS = 1e-12


def _rsqrt(x):
    y = plsc.bitcast(jnp.int32(0x5F3759DF) - (plsc.bitcast(x, jnp.int32) >> 1),
                     jnp.float32)
    for _ in range(3):
        y = y * (1.5 - 0.5 * x * y * y)
    return y


def _sc_body(nbatch, pw, ids_hbm, tts_hbm, word_hbm, pos_hbm, type_hbm,
             lnw_hbm, lnb_hbm, out_hbm, wbuf, posb, tb, wv, bv, idxr, ttr,
             gsem):
    wid = lax.axis_index("c") * NS + lax.axis_index("s")
    p0 = wid * pw
    seq = NW * pw

    pltpu.sync_copy(pos_hbm.at[pl.ds(p0, pw)], posb)
    pltpu.sync_copy(type_hbm, tb)
    pltpu.sync_copy(lnw_hbm, wv)
    pltpu.sync_copy(lnb_hbm, bv)

    def batch_body(b, carry):
        tok0 = b * seq + p0
        pltpu.sync_copy(ids_hbm.at[pl.ds(tok0, pw)], idxr)
        pltpu.sync_copy(tts_hbm.at[pl.ds(tok0, pw)], ttr)
        pltpu.async_copy(word_hbm.at[idxr], wbuf, gsem).wait()

        ttv = ttr[...].astype(jnp.float32)
        lanes = lax.iota(jnp.int32, L)

        def tok_body(i, c):
            ti = jnp.sum(jnp.where(lanes == i, ttv, 0.0))
            acc_s = jnp.zeros((L,), jnp.float32)
            acc_q = jnp.zeros((L,), jnp.float32)
            for h in range(HC):
                col = pl.ds(h * L, L)
                t0c = tb[0, col]
                x = (wbuf[i, col] + posb[i, col]
                     + t0c + ti * (tb[1, col] - t0c))
                wbuf[i, col] = x
                acc_s = acc_s + x
                acc_q = acc_q + x * x
            mu = jnp.sum(acc_s) * (1.0 / H)
            ex2 = jnp.sum(acc_q) * (1.0 / H)
            var = ex2 - mu * mu
            rstdv = _rsqrt(jnp.broadcast_to(var + EPS, (L,)))
            muv = jnp.broadcast_to(mu, (L,))
            for h in range(HC):
                col = pl.ds(h * L, L)
                a = rstdv * wv[col]
                wbuf[i, col] = (wbuf[i, col] - muv) * a + bv[col]
            return c

        lax.fori_loop(0, pw, tok_body, 0, unroll=False)
        pltpu.sync_copy(wbuf, out_hbm.at[pl.ds(tok0, pw)])
        return carry

    lax.fori_loop(0, nbatch, batch_body, 0, unroll=False)


def kernel(input_ids, token_type_ids, word_emb, pos_emb, type_emb, ln_w, ln_b):
    b, s = input_ids.shape
    h = word_emb.shape[1]
    assert h == H and s % NW == 0
    pw = s // NW
    ids = input_ids.reshape(-1).astype(jnp.int32)
    tts = token_type_ids.reshape(-1).astype(jnp.int32)

    mesh = plsc.VectorSubcoreMesh(core_axis_name="c", subcore_axis_name="s",
                                  num_cores=NC, num_subcores=NS)
    run = pl.kernel(
        functools.partial(_sc_body, b, pw),
        out_type=jax.ShapeDtypeStruct((b * s, h), jnp.float32),
        mesh=mesh,
        compiler_params=pltpu.CompilerParams(needs_layout_passes=False),
        scratch_types=[
            pltpu.VMEM((pw, h), jnp.float32),
            pltpu.VMEM((pw, h), jnp.float32),
            pltpu.VMEM((2, h), jnp.float32),
            pltpu.VMEM((h,), jnp.float32),
            pltpu.VMEM((h,), jnp.float32),
            pltpu.VMEM((pw,), jnp.int32),
            pltpu.VMEM((pw,), jnp.int32),
            pltpu.SemaphoreType.DMA,
        ],
    )
    out = run(ids, tts, word_emb, pos_emb, type_emb, ln_w, ln_b)
    return out.reshape(b, s, h)


# speedup vs baseline: 1.0839x; 1.0839x over previous
import functools
import jax
import jax.numpy as jnp
from jax import lax
from jax.experimental import pallas as pl
from jax.experimental.pallas import tpu as pltpu
from jax.experimental.pallas import tpu_sc as plsc

H = 768
L = 16
NC, NS = 2, 16
NW = NC * NS
HC = H // L
NBUF = 4
EPS = 1e-12


def _rsqrt(x):
    y = plsc.bitcast(jnp.int32(0x5F3759DF) - (plsc.bitcast(x, jnp.int32) >> 1),
                     jnp.float32)
    for _ in range(3):
        y = y * (1.5 - 0.5 * x * y * y)
    return y


def _sc_body(nbatch, pw, ids_hbm, tts_hbm, word_hbm, pos_hbm, type_hbm,
             lnw_hbm, lnb_hbm, out_hbm, wbuf, posb, tb, wv, bv, idxall,
             ttall, gsem, osem):
    wid = lax.axis_index("c") * NS + lax.axis_index("s")
    p0 = wid * pw
    seq = NW * pw
    lanes = lax.iota(jnp.int32, L)

    pltpu.sync_copy(ids_hbm.at[:, pl.ds(p0, pw)], idxall)
    pltpu.sync_copy(tts_hbm.at[:, pl.ds(p0, pw)], ttall)
    pltpu.sync_copy(pos_hbm.at[pl.ds(p0, pw)], posb)
    pltpu.sync_copy(type_hbm, tb)
    pltpu.sync_copy(lnw_hbm, wv)
    pltpu.sync_copy(lnb_hbm, bv)

    for h in range(HC):
        col = pl.ds(h * L, L)
        tb[1, col] = tb[1, col] - tb[0, col]

    def fold_body(i, c):
        for h in range(HC):
            col = pl.ds(h * L, L)
            posb[i, col] = posb[i, col] + tb[0, col]
        return c

    lax.fori_loop(0, pw, fold_body, 0, unroll=False)

    def issue_gather(b, s):
        idxv = idxall[b, :]
        pltpu.async_copy(word_hbm.at[idxv], wbuf.at[s], gsem.at[s])

    def wait_gather(s):
        pltpu.make_async_copy(word_hbm.at[pl.ds(0, pw)], wbuf.at[s],
                              gsem.at[s]).wait()

    def wait_out(s):
        pltpu.make_async_copy(wbuf.at[s], out_hbm.at[pl.ds(0, pw)],
                              osem.at[s]).wait()

    def compute_chunk(b, s):
        ttv = ttall[b, :].astype(jnp.float32)

        def tok_body(i, c):
            ti = jnp.sum(jnp.where(lanes == i, ttv, 0.0))
            acc_s = jnp.zeros((L,), jnp.float32)
            acc_q = jnp.zeros((L,), jnp.float32)
            for h in range(HC):
                col = pl.ds(h * L, L)
                x = wbuf[s, i, col] + posb[i, col] + ti * tb[1, col]
                wbuf[s, i, col] = x
                acc_s = acc_s + x
                acc_q = acc_q + x * x
            mu = jnp.sum(acc_s) * (1.0 / H)
            ex2 = jnp.sum(acc_q) * (1.0 / H)
            var = ex2 - mu * mu
            rstdv = _rsqrt(jnp.broadcast_to(var + EPS, (L,)))
            muv = jnp.broadcast_to(mu, (L,))
            for h in range(HC):
                col = pl.ds(h * L, L)
                a = rstdv * wv[col]
                wbuf[s, i, col] = (wbuf[s, i, col] - muv) * a + bv[col]
            return c

        lax.fori_loop(0, pw, tok_body, 0, unroll=False)
        pltpu.async_copy(wbuf.at[s], out_hbm.at[pl.ds(b * seq + p0, pw)],
                         osem.at[s])

    issue_gather(0, 0)
    issue_gather(1, 1)

    ngroup = nbatch // NBUF

    def group_body(g, carry):
        for k in range(NBUF):
            b = g * NBUF + k
            if k < 2:
                s2 = k + 2

                @pl.when(g > 0)
                def _():
                    wait_out(s2)

                issue_gather(b + 2, s2)
            else:
                s2 = k - 2

                @pl.when(g < ngroup - 1)
                def _():
                    wait_out(s2)
                    issue_gather(b + 2, s2)

            wait_gather(k)
            compute_chunk(b, k)
        return carry

    lax.fori_loop(0, ngroup, group_body, 0, unroll=False)

    for s in range(NBUF):
        wait_out(s)


def kernel(input_ids, token_type_ids, word_emb, pos_emb, type_emb, ln_w, ln_b):
    b, s = input_ids.shape
    h = word_emb.shape[1]
    assert h == H and s % NW == 0 and b % NBUF == 0
    pw = s // NW
    ids = input_ids.astype(jnp.int32)
    tts = token_type_ids.astype(jnp.int32)

    mesh = plsc.VectorSubcoreMesh(core_axis_name="c", subcore_axis_name="s",
                                  num_cores=NC, num_subcores=NS)
    run = pl.kernel(
        functools.partial(_sc_body, b, pw),
        out_type=jax.ShapeDtypeStruct((b * s, h), jnp.float32),
        mesh=mesh,
        compiler_params=pltpu.CompilerParams(needs_layout_passes=False,
                                             use_tc_tiling_on_sc=False),
        scratch_types=[
            pltpu.VMEM((NBUF, pw, h), jnp.float32),
            pltpu.VMEM((pw, h), jnp.float32),
            pltpu.VMEM((2, h), jnp.float32),
            pltpu.VMEM((h,), jnp.float32),
            pltpu.VMEM((h,), jnp.float32),
            pltpu.VMEM((b, pw), jnp.int32),
            pltpu.VMEM((b, pw), jnp.int32),
            pltpu.SemaphoreType.DMA((NBUF,)),
            pltpu.SemaphoreType.DMA((NBUF,)),
        ],
    )
    out = run(ids, tts, word_emb, pos_emb, type_emb, ln_w, ln_b)
    return out.reshape(b, s, h)
